# baseline (speedup 1.0000x reference)
import jax
import jax.numpy as jnp
from jax.experimental import pallas as pl
from jax.experimental.pallas import tpu as pltpu

_BP = 32


def _dwt_kernel(x_ref, ll_ref, lh_ref, hl_ref, hh_ref):
    xb = x_ref[...]
    t = xb[:, 0]
    u = xb[:, 1]
    s = (t + u) * jnp.float32(0.5)
    m = (t - u) * jnp.float32(0.5)

    lane = jax.lax.broadcasted_iota(jnp.int32, (1, 1, 128), 2)
    perm = jnp.where(lane < 64, 2 * lane, 2 * (lane - 64) + 1)
    lo = lane < 64

    def mix(v):
        sum_chunks = []
        dif_chunks = []
        e = []
        r = []
        for q in range(4):
            vq = v[:, :, q * 128:(q + 1) * 128]
            eq = jnp.take_along_axis(vq, jnp.broadcast_to(perm, vq.shape),
                                     axis=2)
            e.append(eq)
            r.append(pltpu.roll(eq, 64, axis=2))
        for j in range(2):
            q0, q1 = 2 * j, 2 * j + 1
            sum_chunks.append(
                jnp.where(lo, e[q0] + r[q0], e[q1] + r[q1]))
            dif_chunks.append(
                jnp.where(lo, e[q0] - r[q0], r[q1] - e[q1]))
        return (jnp.concatenate(sum_chunks, axis=2),
                jnp.concatenate(dif_chunks, axis=2))

    ll, lh = mix(s)
    hl, hh = mix(m)
    ll_ref[...] = ll
    lh_ref[...] = lh
    hl_ref[...] = hl
    hh_ref[...] = hh


def kernel(x):
    B, H, W, C = x.shape
    RP = B * H // 2
    xt = jnp.transpose(x, (0, 1, 3, 2))
    xv = xt.reshape(RP, 2, C, W)
    out_sds = jax.ShapeDtypeStruct((RP, C, W // 2), x.dtype)
    outs = pl.pallas_call(
        _dwt_kernel,
        grid=(RP // _BP,),
        in_specs=[pl.BlockSpec((_BP, 2, C, W), lambda i: (i, 0, 0, 0))],
        out_specs=[pl.BlockSpec((_BP, C, W // 2), lambda i: (i, 0, 0))] * 4,
        out_shape=[out_sds] * 4,
        compiler_params=pltpu.CompilerParams(
            dimension_semantics=("parallel",),
        ),
    )(xv)
    H2, W2 = H // 2, W // 2
    return tuple(
        jnp.transpose(o.reshape(B, H2, C, W2), (0, 1, 3, 2)) for o in outs
    )


# speedup vs baseline: 1.2872x; 1.2872x over previous
import jax
import jax.numpy as jnp
from jax.experimental import pallas as pl
from jax.experimental.pallas import tpu as pltpu

_BP = 128


def _dwt_kernel(x_ref, ll_ref, lh_ref, hl_ref, hh_ref):
    xb = x_ref[...]
    t = xb[:, 0]
    u = xb[:, 1]
    s = (t + u) * jnp.float32(0.5)
    m = (t - u) * jnp.float32(0.5)

    lane = jax.lax.broadcasted_iota(jnp.int32, (1, 1, 128), 2)
    perm = jnp.where(lane < 64, 2 * lane, 2 * (lane - 64) + 1)
    lo = lane < 64

    def mix(v):
        sum_chunks = []
        dif_chunks = []
        e = []
        r = []
        for q in range(4):
            vq = v[:, :, q * 128:(q + 1) * 128]
            eq = jnp.take_along_axis(vq, jnp.broadcast_to(perm, vq.shape),
                                     axis=2)
            e.append(eq)
            r.append(pltpu.roll(eq, 64, axis=2))
        for j in range(2):
            q0, q1 = 2 * j, 2 * j + 1
            sum_chunks.append(
                jnp.where(lo, e[q0] + r[q0], e[q1] + r[q1]))
            dif_chunks.append(
                jnp.where(lo, e[q0] - r[q0], r[q1] - e[q1]))
        return (jnp.concatenate(sum_chunks, axis=2),
                jnp.concatenate(dif_chunks, axis=2))

    ll, lh = mix(s)
    hl, hh = mix(m)
    ll_ref[...] = ll
    lh_ref[...] = lh
    hl_ref[...] = hl
    hh_ref[...] = hh


def kernel(x):
    B, H, W, C = x.shape
    RP = B * H // 2
    xt = jnp.transpose(x, (0, 1, 3, 2))
    xv = xt.reshape(RP, 2, C, W)
    out_sds = jax.ShapeDtypeStruct((RP, C, W // 2), x.dtype)
    outs = pl.pallas_call(
        _dwt_kernel,
        grid=(RP // _BP,),
        in_specs=[pl.BlockSpec((_BP, 2, C, W), lambda i: (i, 0, 0, 0))],
        out_specs=[pl.BlockSpec((_BP, C, W // 2), lambda i: (i, 0, 0))] * 4,
        out_shape=[out_sds] * 4,
        compiler_params=pltpu.CompilerParams(
            dimension_semantics=("parallel",),
        ),
    )(xv)
    H2, W2 = H // 2, W // 2
    return tuple(
        jnp.transpose(o.reshape(B, H2, C, W2), (0, 1, 3, 2)) for o in outs
    )
